# Initial kernel scaffold; baseline (speedup 1.0000x reference)
import jax
import jax.numpy as jnp
from jax.experimental import pallas as pl


def kernel(inputs, embedding):
    raise NotImplementedError("write your pallas kernel here")


# speedup vs baseline: 1.5475x; 1.5475x over previous
import functools

import jax
import jax.numpy as jnp
from jax import lax
from jax.experimental import pallas as pl
from jax.experimental.pallas import tpu as pltpu
from jax.experimental.pallas import tpu_sc as plsc

BATCH = 16384
FIELDS = 26
FEATURES = 32
TOTAL = BATCH * FIELDS

NUM_CORES = 2
NUM_SUBCORES = 16
NUM_WORKERS = NUM_CORES * NUM_SUBCORES
PER_WORKER = TOTAL // NUM_WORKERS
CHUNK = 1024
NUM_CHUNKS = PER_WORKER // CHUNK


@functools.partial(
    pl.kernel,
    mesh=plsc.VectorSubcoreMesh(core_axis_name="c", subcore_axis_name="s"),
    compiler_params=pltpu.CompilerParams(use_tc_tiling_on_sc=False),
    out_type=jax.ShapeDtypeStruct((TOTAL, FEATURES), jnp.float32),
    scratch_types=[
        pltpu.VMEM((CHUNK,), jnp.int32),
        pltpu.VMEM((CHUNK, FEATURES), jnp.float32),
        pltpu.SemaphoreType.DMA,
    ],
)
def _sc_gather(idx_hbm, table_hbm, out_hbm, idx_v, rows_v, sem):
    wid = lax.axis_index("s") * NUM_CORES + lax.axis_index("c")
    base = wid * PER_WORKER

    def body(c, carry):
        off = base + c * CHUNK
        pltpu.sync_copy(idx_hbm.at[pl.ds(off, CHUNK)], idx_v)
        pltpu.async_copy(table_hbm.at[idx_v], rows_v, sem).wait()
        pltpu.sync_copy(rows_v, out_hbm.at[pl.ds(off, CHUNK)])
        return carry

    lax.fori_loop(0, NUM_CHUNKS, body, 0)


def kernel(inputs, embedding):
    flat_idx = inputs.reshape(-1)
    out = _sc_gather(flat_idx, embedding)
    return out.reshape(BATCH, FIELDS, FEATURES)


# speedup vs baseline: 1.5770x; 1.0190x over previous
import functools

import jax
import jax.numpy as jnp
from jax import lax
from jax.experimental import pallas as pl
from jax.experimental.pallas import tpu as pltpu
from jax.experimental.pallas import tpu_sc as plsc

BATCH = 16384
FIELDS = 26
FEATURES = 32
TOTAL = BATCH * FIELDS

NUM_CORES = 2
NUM_SUBCORES = 16
NUM_WORKERS = NUM_CORES * NUM_SUBCORES
PER_WORKER = TOTAL // NUM_WORKERS
CHUNK = 1024
NUM_CHUNKS = PER_WORKER // CHUNK
NBUF = 3
LEAD = 2


@functools.partial(
    pl.kernel,
    mesh=plsc.VectorSubcoreMesh(core_axis_name="c", subcore_axis_name="s"),
    compiler_params=pltpu.CompilerParams(use_tc_tiling_on_sc=False),
    out_type=jax.ShapeDtypeStruct((TOTAL, FEATURES), jnp.float32),
    scratch_types=[
        pltpu.VMEM((PER_WORKER,), jnp.int32),
        pltpu.VMEM((NBUF, CHUNK, FEATURES), jnp.float32),
        pltpu.SemaphoreType.DMA((NBUF,)),
        pltpu.SemaphoreType.DMA((NBUF,)),
        pltpu.SemaphoreType.DMA,
    ],
)
def _sc_gather(idx_hbm, table_hbm, out_hbm, idx_v, rows_v, gsem, wsem, isem):
    wid = lax.axis_index("s") * NUM_CORES + lax.axis_index("c")
    base = wid * PER_WORKER

    pltpu.async_copy(idx_hbm.at[pl.ds(base, PER_WORKER)], idx_v, isem).wait()

    gathers = [None] * NUM_CHUNKS
    writes = [None] * NUM_CHUNKS
    for step in range(NUM_CHUNKS + LEAD):
        if step < NUM_CHUNKS:
            c = step
            b = c % NBUF
            if c >= NBUF:
                writes[c - NBUF].wait()
            gathers[c] = pltpu.async_copy(
                table_hbm.at[idx_v.at[pl.ds(c * CHUNK, CHUNK)]],
                rows_v.at[b],
                gsem.at[b],
            )
        if step >= LEAD:
            j = step - LEAD
            b = j % NBUF
            gathers[j].wait()
            writes[j] = pltpu.async_copy(
                rows_v.at[b],
                out_hbm.at[pl.ds(base + j * CHUNK, CHUNK)],
                wsem.at[b],
            )
    for j in range(max(0, NUM_CHUNKS - NBUF), NUM_CHUNKS):
        writes[j].wait()


def kernel(inputs, embedding):
    flat_idx = inputs.reshape(-1)
    out = _sc_gather(flat_idx, embedding)
    return out.reshape(BATCH, FIELDS, FEATURES)


# speedup vs baseline: 1.6287x; 1.0328x over previous
import functools

import jax
import jax.numpy as jnp
from jax import lax
from jax.experimental import pallas as pl
from jax.experimental.pallas import tpu as pltpu
from jax.experimental.pallas import tpu_sc as plsc

BATCH = 16384
FIELDS = 26
FEATURES = 32

NUM_CORES = 2
NUM_SUBCORES = 16
NUM_WORKERS = NUM_CORES * NUM_SUBCORES
BTILES = BATCH // 128
BT_PER_WORKER = BTILES // NUM_WORKERS
NUNITS = BT_PER_WORKER * FIELDS
NBUF = 4


@functools.partial(
    pl.kernel,
    mesh=plsc.VectorSubcoreMesh(core_axis_name="c", subcore_axis_name="s"),
    compiler_params=pltpu.CompilerParams(use_tc_tiling_on_sc=False),
    out_type=jax.ShapeDtypeStruct((FIELDS, BATCH, FEATURES), jnp.float32),
    scratch_types=[
        pltpu.VMEM((NBUF, 128), jnp.int32),
        pltpu.VMEM((NBUF, 128, FEATURES), jnp.float32),
        pltpu.VMEM((128, FEATURES), jnp.float32),
        pltpu.SemaphoreType.DMA((NBUF,)),
        pltpu.SemaphoreType.DMA((NBUF,)),
        pltpu.SemaphoreType.DMA((NBUF,)),
    ],
)
def _sc_gather(idxt_hbm, table_hbm, out_hbm, list_v, rows_v, dummy_v,
               isem, gsem, wsem):
    wid = lax.axis_index("s") * NUM_CORES + lax.axis_index("c")
    t0 = wid * BT_PER_WORKER

    def start_idx(u, b):
        t_loc = u // FIELDS
        f = u % FIELDS
        return pltpu.async_copy(
            idxt_hbm.at[pl.ds(f * BATCH + (t0 + t_loc) * 128, 128)],
            list_v.at[b],
            isem.at[b],
        )

    def start_gather(b):
        return pltpu.async_copy(
            table_hbm.at[list_v.at[b]], rows_v.at[b], gsem.at[b]
        )

    def start_write(u, b):
        t_loc = u // FIELDS
        f = u % FIELDS
        return pltpu.async_copy(
            rows_v.at[b],
            out_hbm.at[f, pl.ds((t0 + t_loc) * 128, 128)],
            wsem.at[b],
        )

    def drain_write(b):
        pltpu.make_async_copy(
            dummy_v, out_hbm.at[0, pl.ds(0, 128)], wsem.at[b]
        ).wait()

    def body(i, carry):
        u0 = i * NBUF
        idx_dmas = []
        for b in range(NBUF):
            @pl.when(i > 0)
            def _():
                drain_write(b)
            idx_dmas.append(start_idx(u0 + b, b))
        gathers = []
        for b in range(NBUF):
            idx_dmas[b].wait()
            gathers.append(start_gather(b))
        for b in range(NBUF):
            gathers[b].wait()
            start_write(u0 + b, b)
        return carry

    lax.fori_loop(0, NUNITS // NBUF, body, 0)
    for b in range(NBUF):
        drain_write(b)


def kernel(inputs, embedding):
    idxt = inputs.T.reshape(-1)
    out3 = _sc_gather(idxt, embedding)
    return jnp.swapaxes(out3, 0, 1)


# speedup vs baseline: 1.6454x; 1.0102x over previous
import functools

import jax
import jax.numpy as jnp
from jax import lax
from jax.experimental import pallas as pl
from jax.experimental.pallas import tpu as pltpu
from jax.experimental.pallas import tpu_sc as plsc

BATCH = 16384
FIELDS = 26
FEATURES = 32

NUM_CORES = 2
NUM_SUBCORES = 16
NUM_WORKERS = NUM_CORES * NUM_SUBCORES
BTILES = BATCH // 128
BT_PER_WORKER = BTILES // NUM_WORKERS
NUNITS = FIELDS
ROWS = BT_PER_WORKER * 128
NBUF = 2


@functools.partial(
    pl.kernel,
    mesh=plsc.VectorSubcoreMesh(core_axis_name="c", subcore_axis_name="s"),
    compiler_params=pltpu.CompilerParams(use_tc_tiling_on_sc=False),
    out_type=jax.ShapeDtypeStruct((FIELDS, BATCH, FEATURES), jnp.float32),
    scratch_types=[
        pltpu.VMEM((NBUF, ROWS), jnp.int32),
        pltpu.VMEM((NBUF, ROWS, FEATURES), jnp.float32),
        pltpu.VMEM((ROWS, FEATURES), jnp.float32),
        pltpu.SemaphoreType.DMA((NBUF,)),
        pltpu.SemaphoreType.DMA((NBUF,)),
        pltpu.SemaphoreType.DMA((NBUF,)),
    ],
)
def _sc_gather(idxt_hbm, table_hbm, out_hbm, list_v, rows_v, dummy_v,
               isem, gsem, wsem):
    wid = lax.axis_index("s") * NUM_CORES + lax.axis_index("c")
    t0 = wid * BT_PER_WORKER

    def start_idx(u, b):
        return pltpu.async_copy(
            idxt_hbm.at[pl.ds(u * BATCH + t0 * 128, ROWS)],
            list_v.at[b],
            isem.at[b],
        )

    def start_gather(b):
        return pltpu.async_copy(
            table_hbm.at[list_v.at[b]], rows_v.at[b], gsem.at[b]
        )

    def start_write(u, b):
        return pltpu.async_copy(
            rows_v.at[b],
            out_hbm.at[u, pl.ds(t0 * 128, ROWS)],
            wsem.at[b],
        )

    def drain_write(b):
        pltpu.make_async_copy(
            dummy_v, out_hbm.at[0, pl.ds(0, ROWS)], wsem.at[b]
        ).wait()

    def body(i, carry):
        u0 = i * NBUF
        idx_dmas = []
        for b in range(NBUF):
            @pl.when(i > 0)
            def _():
                drain_write(b)
            idx_dmas.append(start_idx(u0 + b, b))
        gathers = []
        for b in range(NBUF):
            idx_dmas[b].wait()
            gathers.append(start_gather(b))
        for b in range(NBUF):
            gathers[b].wait()
            start_write(u0 + b, b)
        return carry

    lax.fori_loop(0, NUNITS // NBUF, body, 0)
    for b in range(NBUF):
        drain_write(b)


def kernel(inputs, embedding):
    idxt = inputs.T.reshape(-1)
    out3 = _sc_gather(idxt, embedding)
    return jnp.swapaxes(out3, 0, 1)


# speedup vs baseline: 1.6715x; 1.0159x over previous
import functools

import jax
import jax.numpy as jnp
from jax import lax
from jax.experimental import pallas as pl
from jax.experimental.pallas import tpu as pltpu
from jax.experimental.pallas import tpu_sc as plsc

BATCH = 16384
FIELDS = 26
FEATURES = 32

NUM_CORES = 2
NUM_SUBCORES = 16
NUM_WORKERS = NUM_CORES * NUM_SUBCORES
BTILES = BATCH // 128
BT_PER_WORKER = BTILES // NUM_WORKERS
NUNITS = FIELDS
ROWS = BT_PER_WORKER * 128
NBUF = 2


@functools.partial(
    pl.kernel,
    mesh=plsc.VectorSubcoreMesh(core_axis_name="c", subcore_axis_name="s"),
    compiler_params=pltpu.CompilerParams(use_tc_tiling_on_sc=False),
    out_type=jax.ShapeDtypeStruct((FIELDS, BATCH, FEATURES), jnp.float32),
    scratch_types=[
        pltpu.VMEM((NBUF, ROWS), jnp.int32),
        pltpu.VMEM((NBUF, ROWS, FEATURES), jnp.float32),
        pltpu.VMEM((ROWS, FEATURES), jnp.float32),
        pltpu.SemaphoreType.DMA((NBUF,)),
        pltpu.SemaphoreType.DMA((NBUF,)),
        pltpu.SemaphoreType.DMA((NBUF,)),
    ],
)
def _sc_gather(idxt_hbm, table_hbm, out_hbm, list_v, rows_v, dummy_v,
               isem, gsem, wsem):
    wid = lax.axis_index("s") * NUM_CORES + lax.axis_index("c")
    t0 = wid * BT_PER_WORKER

    def start_idx(u, b):
        return pltpu.async_copy(
            idxt_hbm.at[pl.ds(u * BATCH + t0 * 128, ROWS)],
            list_v.at[b],
            isem.at[b],
        )

    def start_gather(b):
        return pltpu.async_copy(
            table_hbm.at[list_v.at[b]], rows_v.at[b], gsem.at[b]
        )

    def start_write(u, b):
        return pltpu.async_copy(
            rows_v.at[b],
            out_hbm.at[u, pl.ds(t0 * 128, ROWS)],
            wsem.at[b],
        )

    def drain_write(b):
        pltpu.make_async_copy(
            dummy_v, out_hbm.at[0, pl.ds(0, ROWS)], wsem.at[b]
        ).wait()

    def scale_list(b):
        for g in range(ROWS // 16):
            v = list_v[b, pl.ds(g * 16, 16)]
            list_v[b, pl.ds(g * 16, 16)] = v * 4

    def body(i, carry):
        u0 = i * NBUF
        idx_dmas = []
        for b in range(NBUF):
            @pl.when(i > 0)
            def _():
                drain_write(b)
            idx_dmas.append(start_idx(u0 + b, b))
        gathers = []
        for b in range(NBUF):
            idx_dmas[b].wait()
            scale_list(b)
            gathers.append(start_gather(b))
        for b in range(NBUF):
            gathers[b].wait()
            start_write(u0 + b, b)
        return carry

    lax.fori_loop(0, NUNITS // NBUF, body, 0)
    for b in range(NBUF):
        drain_write(b)


def kernel(inputs, embedding):
    idxt = inputs.T.reshape(-1)
    emb_pad = jnp.concatenate(
        [embedding, jnp.zeros((1000000, 96), jnp.float32)], axis=1
    ).reshape(4000000, 32)
    out3 = _sc_gather(idxt, emb_pad)
    return jnp.swapaxes(out3, 0, 1)
